# Initial kernel scaffold; baseline (speedup 1.0000x reference)
import jax
import jax.numpy as jnp
from jax.experimental import pallas as pl


def kernel(x, edge_index, W, a_left, a_right):
    raise NotImplementedError("write your pallas kernel here")


# speedup vs baseline: 7.0271x; 7.0271x over previous
import functools

import jax
import jax.numpy as jnp
from jax import lax
from jax.experimental import pallas as pl
from jax.experimental.pallas import tpu as pltpu
from jax.experimental.pallas import tpu_sc as plsc

N_NODES = 10000
N_EDGES = 160000
D_IN = 256
H = 8
C = 64
NEG = 0.2

E_TOT = N_EDGES + N_NODES
N_SUB = 16
SB = 128
SB_PER_TILE = 84
EPT = SB * SB_PER_TILE
E_PAD = EPT * N_SUB
N_PAD = 10112
ROWS_PER_TILE = N_PAD // N_SUB
AW = 80

BN = 400


def _mm_body(x_ref, w_ref, wl_ref, wr_ref, wh_ref, el_ref, er_ref):
    xb = x_ref[...]
    wh_ref[...] = jnp.dot(xb, w_ref[...], preferred_element_type=jnp.float32)
    el_ref[...] = jnp.dot(xb, wl_ref[...], preferred_element_type=jnp.float32)
    er_ref[...] = jnp.dot(xb, wr_ref[...], preferred_element_type=jnp.float32)


def _project(x, W, Wl, Wr):
    nblk = N_NODES // BN
    return pl.pallas_call(
        _mm_body,
        grid=(nblk,),
        in_specs=[
            pl.BlockSpec((BN, D_IN), lambda i: (i, 0)),
            pl.BlockSpec((D_IN, H * C), lambda i: (0, 0)),
            pl.BlockSpec((D_IN, H), lambda i: (0, 0)),
            pl.BlockSpec((D_IN, H), lambda i: (0, 0)),
        ],
        out_specs=[
            pl.BlockSpec((BN, H * C), lambda i: (i, 0)),
            pl.BlockSpec((BN, H), lambda i: (i, 0)),
            pl.BlockSpec((BN, H), lambda i: (i, 0)),
        ],
        out_shape=[
            jax.ShapeDtypeStruct((N_NODES, H * C), jnp.float32),
            jax.ShapeDtypeStruct((N_NODES, H), jnp.float32),
            jax.ShapeDtypeStruct((N_NODES, H), jnp.float32),
        ],
    )(x, W, Wl, Wr)


def _make_edge_kernel(kcall: int):
    mesh = plsc.VectorSubcoreMesh(core_axis_name="c", subcore_axis_name="s")

    @functools.partial(
        pl.kernel,
        out_type=jax.ShapeDtypeStruct((4, N_PAD, AW), jnp.float32),
        mesh=mesh,
        compiler_params=pltpu.CompilerParams(
            needs_layout_passes=False, use_tc_tiling_on_sc=False),
        scratch_types=[
            pltpu.VMEM((EPT,), jnp.int32),
            pltpu.VMEM((EPT,), jnp.int32),
            pltpu.VMEM((N_NODES,), jnp.float32),
            pltpu.VMEM((N_NODES,), jnp.float32),
            pltpu.VMEM((16,), jnp.float32),
            pltpu.VMEM((SB,), jnp.int32),
            pltpu.VMEM((SB,), jnp.float32),
            pltpu.VMEM((SB,), jnp.int32),
            pltpu.VMEM((SB, C), jnp.float32),
            pltpu.VMEM((SB, AW), jnp.float32),
            pltpu.VMEM_SHARED((N_PAD, AW), jnp.float32),
            pltpu.SemaphoreType.DMA,
        ],
    )
    def edge_kernel(whr, elt, ert, srcp, dstp, mvec, zrows, out,
                    src_v, dst_v, el_v, er_v, m_v,
                    idx_v, w_v, dstb_v, rows_v, outb_v, acc, sem):
        cid = lax.axis_index("c")
        sid = lax.axis_index("s")
        base = sid * EPT
        h0 = 4 * kcall + 2 * cid
        r0 = sid * ROWS_PER_TILE

        pltpu.sync_copy(srcp.at[pl.ds(base, EPT)], src_v)
        pltpu.sync_copy(dstp.at[pl.ds(base, EPT)], dst_v)
        pltpu.sync_copy(mvec, m_v)

        z16 = jnp.zeros((16,), jnp.float32)
        for r in range(SB):
            outb_v[r, pl.ds(C, 16)] = z16

        iota16 = lax.iota(jnp.int32, 16)
        mval = m_v[...]
        neg_inf = jnp.float32(-3e38)

        for hl in range(2):
            hg = h0 + hl
            pltpu.sync_copy(elt.at[pl.ds(hg * N_NODES, N_NODES)], el_v)
            pltpu.sync_copy(ert.at[pl.ds(hg * N_NODES, N_NODES)], er_v)
            pltpu.sync_copy(zrows.at[pl.ds(r0, ROWS_PER_TILE)],
                            acc.at[pl.ds(r0, ROWS_PER_TILE)])
            mh = jnp.zeros((16,), jnp.float32) + jnp.max(
                jnp.where(iota16 == hg, mval, neg_inf))
            plsc.subcore_barrier()

            def sb_body(sb, carry, mh=mh, hg=hg):
                off = sb * SB
                for j in range(SB // 16):
                    s16 = src_v[pl.ds(off + j * 16, 16)]
                    d16 = dst_v[pl.ds(off + j * 16, 16)]
                    e = (plsc.load_gather(el_v, [s16])
                         + plsc.load_gather(er_v, [d16]))
                    e = jnp.where(e > 0, e, NEG * e)
                    w = jnp.exp(e - mh)
                    gid = base + off + j * 16 + iota16
                    w = jnp.where(gid < E_TOT, w, 0.0)
                    idx_v[pl.ds(j * 16, 16)] = s16 * H + hg
                    w_v[pl.ds(j * 16, 16)] = w
                    dstb_v[pl.ds(j * 16, 16)] = d16
                pltpu.async_copy(whr.at[idx_v], rows_v, sem).wait()
                for j in range(SB // 16):
                    w16 = w_v[pl.ds(j * 16, 16)]
                    ridx = j * 16 + iota16

                    def cbody(c, _, ridx=ridx, w16=w16):
                        cc = jnp.full((16,), c, jnp.int32)
                        v = plsc.load_gather(rows_v, [ridx, cc])
                        plsc.store_scatter(outb_v, [ridx, cc], v * w16)
                        return 0

                    lax.fori_loop(0, C, cbody, 0)
                    plsc.store_scatter(
                        outb_v, [ridx, jnp.full((16,), C, jnp.int32)], w16)
                pltpu.sync_copy(outb_v, acc.at[dstb_v], add=True)
                return carry

            lax.fori_loop(0, SB_PER_TILE, sb_body, 0)

            plsc.subcore_barrier()
            pltpu.sync_copy(acc.at[pl.ds(r0, ROWS_PER_TILE)],
                            out.at[2 * cid + hl, pl.ds(r0, ROWS_PER_TILE)])
            plsc.subcore_barrier()

    return edge_kernel


_EDGE_K0 = _make_edge_kernel(0)
_EDGE_K1 = _make_edge_kernel(1)


def _norm_body(acc_ref, out_ref):
    for h in range(H):
        a = acc_ref[h]
        out_ref[:, h * C:(h + 1) * C] = a[:, :C] / (a[:, C:C + 1] + 1e-16)


def _normalize(acc8):
    nblk = N_NODES // BN
    return pl.pallas_call(
        _norm_body,
        grid=(nblk,),
        in_specs=[pl.BlockSpec((H, BN, AW), lambda i: (0, i, 0))],
        out_specs=pl.BlockSpec((BN, H * C), lambda i: (i, 0)),
        out_shape=jax.ShapeDtypeStruct((N_NODES, H * C), jnp.float32),
    )(acc8)


def kernel(x, edge_index, W, a_left, a_right):
    W3 = W.reshape(D_IN, H, C)
    Wl = jnp.einsum("ihc,hc->ih", W3, a_left)
    Wr = jnp.einsum("ihc,hc->ih", W3, a_right)

    wh, el, er = _project(x, W, Wl, Wr)
    whr = wh.reshape(N_NODES * H, C)
    elt = el.T.reshape(H * N_NODES)
    ert = er.T.reshape(H * N_NODES)

    m = el.max(axis=0) + er.max(axis=0)
    m = jnp.where(m > 0, m, NEG * m)
    m16 = jnp.pad(m, (0, 16 - H)).astype(jnp.float32)

    loop = jnp.arange(N_NODES, dtype=jnp.int32)
    src = jnp.concatenate([edge_index[0].astype(jnp.int32), loop])
    dst = jnp.concatenate([edge_index[1].astype(jnp.int32), loop])
    srcp = jnp.pad(src, (0, E_PAD - E_TOT))
    dstp = jnp.pad(dst, (0, E_PAD - E_TOT))

    zrows = jnp.zeros((N_PAD, AW), jnp.float32)

    acc_lo = _EDGE_K0(whr, elt, ert, srcp, dstp, m16, zrows)
    acc_hi = _EDGE_K1(whr, elt, ert, srcp, dstp, m16, zrows)
    acc8 = jnp.concatenate([acc_lo, acc_hi], axis=0)[:, :N_NODES, :]

    return _normalize(acc8)


# speedup vs baseline: 8.0525x; 1.1459x over previous
import functools

import jax
import jax.numpy as jnp
from jax import lax
from jax.experimental import pallas as pl
from jax.experimental.pallas import tpu as pltpu
from jax.experimental.pallas import tpu_sc as plsc

N_NODES = 10000
N_EDGES = 160000
D_IN = 256
H = 8
C = 64
NEG = 0.2

E_TOT = N_EDGES + N_NODES
N_SUB = 16
SB = 128
SB_PER_TILE = 84
EPT = SB * SB_PER_TILE
E_PAD = EPT * N_SUB
N_PAD = 10112
ROWS_PER_TILE = N_PAD // N_SUB
AW = 80

BN = 400


def _mm_body(x_ref, w_ref, wl_ref, wr_ref, wh_ref, el_ref, er_ref):
    xb = x_ref[...]
    wh_ref[...] = jnp.dot(xb, w_ref[...], preferred_element_type=jnp.float32)
    el_ref[...] = jnp.dot(xb, wl_ref[...], preferred_element_type=jnp.float32)
    er_ref[...] = jnp.dot(xb, wr_ref[...], preferred_element_type=jnp.float32)


def _project(x, W, Wl, Wr):
    nblk = N_NODES // BN
    return pl.pallas_call(
        _mm_body,
        grid=(nblk,),
        in_specs=[
            pl.BlockSpec((BN, D_IN), lambda i: (i, 0)),
            pl.BlockSpec((D_IN, H * C), lambda i: (0, 0)),
            pl.BlockSpec((D_IN, H), lambda i: (0, 0)),
            pl.BlockSpec((D_IN, H), lambda i: (0, 0)),
        ],
        out_specs=[
            pl.BlockSpec((BN, H * C), lambda i: (i, 0)),
            pl.BlockSpec((BN, H), lambda i: (i, 0)),
            pl.BlockSpec((BN, H), lambda i: (i, 0)),
        ],
        out_shape=[
            jax.ShapeDtypeStruct((N_NODES, H * C), jnp.float32),
            jax.ShapeDtypeStruct((N_NODES, H), jnp.float32),
            jax.ShapeDtypeStruct((N_NODES, H), jnp.float32),
        ],
    )(x, W, Wl, Wr)


def _make_edge_kernel(kcall: int):
    mesh = plsc.VectorSubcoreMesh(core_axis_name="c", subcore_axis_name="s")

    @functools.partial(
        pl.kernel,
        out_type=jax.ShapeDtypeStruct((4, N_PAD, AW), jnp.float32),
        mesh=mesh,
        compiler_params=pltpu.CompilerParams(
            needs_layout_passes=False, use_tc_tiling_on_sc=False),
        scratch_types=[
            pltpu.VMEM((EPT + SB,), jnp.int32),
            pltpu.VMEM((EPT + SB,), jnp.int32),
            pltpu.VMEM((N_NODES,), jnp.float32),
            pltpu.VMEM((N_NODES,), jnp.float32),
            pltpu.VMEM((16,), jnp.float32),
            pltpu.VMEM((SB,), jnp.int32),
            pltpu.VMEM((SB,), jnp.float32),
            pltpu.VMEM((SB,), jnp.int32),
            pltpu.VMEM((SB, C), jnp.float32),
            pltpu.VMEM((SB,), jnp.int32),
            pltpu.VMEM((SB,), jnp.float32),
            pltpu.VMEM((SB,), jnp.int32),
            pltpu.VMEM((SB, C), jnp.float32),
            pltpu.VMEM((SB, AW), jnp.float32),
            pltpu.VMEM_SHARED((N_PAD, AW), jnp.float32),
            pltpu.SemaphoreType.DMA,
            pltpu.SemaphoreType.DMA,
        ],
    )
    def edge_kernel(whr, elt, ert, srcp, dstp, mvec, zrows, out,
                    src_v, dst_v, el_v, er_v, m_v,
                    idx_a, w_a, dst_a, rows_a,
                    idx_b, w_b, dst_b, rows_b,
                    outb_v, acc, sem_a, sem_b):
        cid = lax.axis_index("c")
        sid = lax.axis_index("s")
        base = sid * EPT
        h0 = 4 * kcall + 2 * cid
        r0 = sid * ROWS_PER_TILE

        pltpu.sync_copy(srcp.at[pl.ds(base, EPT)], src_v.at[pl.ds(0, EPT)])
        pltpu.sync_copy(dstp.at[pl.ds(base, EPT)], dst_v.at[pl.ds(0, EPT)])
        pltpu.sync_copy(mvec, m_v)

        z16 = jnp.zeros((16,), jnp.float32)
        for r in range(SB):
            outb_v[r, pl.ds(C, 16)] = z16

        iota16 = lax.iota(jnp.int32, 16)
        mval = m_v[...]
        neg_inf = jnp.float32(-3e38)

        def fill_fire(sb, mh, hg, idx_v, w_v, dstb_v, rows_v, sem):
            off = sb * SB
            for j in range(SB // 16):
                s16 = jnp.clip(src_v[pl.ds(off + j * 16, 16)], 0, N_NODES - 1)
                d16 = jnp.clip(dst_v[pl.ds(off + j * 16, 16)], 0, N_NODES - 1)
                e = (plsc.load_gather(el_v, [s16])
                     + plsc.load_gather(er_v, [d16]))
                e = jnp.where(e > 0, e, NEG * e)
                w = jnp.exp(e - mh)
                gid = base + off + j * 16 + iota16
                w = jnp.where(gid < E_TOT, w, 0.0)
                idx_v[pl.ds(j * 16, 16)] = s16 * H + hg
                w_v[pl.ds(j * 16, 16)] = w
                dstb_v[pl.ds(j * 16, 16)] = d16
            pltpu.make_async_copy(whr.at[idx_v], rows_v, sem).start()

        def drain_process(idx_v, w_v, dstb_v, rows_v, sem):
            pltpu.make_async_copy(whr.at[idx_v], rows_v, sem).wait()
            for j in range(SB // 16):
                w16 = w_v[pl.ds(j * 16, 16)]
                ridx = j * 16 + iota16

                def cbody(ci, _, ridx=ridx, w16=w16):
                    for u in range(4):
                        cc = jnp.zeros((16,), jnp.int32) + (ci * 4 + u)
                        v = plsc.load_gather(rows_v, [ridx, cc])
                        plsc.store_scatter(outb_v, [ridx, cc], v * w16)
                    return 0

                lax.fori_loop(0, C // 4, cbody, 0)
                plsc.store_scatter(
                    outb_v, [ridx, jnp.full((16,), C, jnp.int32)], w16)
            pltpu.sync_copy(outb_v, acc.at[dstb_v], add=True)

        bufs_a = (idx_a, w_a, dst_a, rows_a, sem_a)
        bufs_b = (idx_b, w_b, dst_b, rows_b, sem_b)

        def head_body(hl, carry):
            hg = h0 + hl
            pltpu.sync_copy(elt.at[pl.ds(hg * N_NODES, N_NODES)], el_v)
            pltpu.sync_copy(ert.at[pl.ds(hg * N_NODES, N_NODES)], er_v)
            pltpu.sync_copy(zrows.at[pl.ds(r0, ROWS_PER_TILE)],
                            acc.at[pl.ds(r0, ROWS_PER_TILE)])
            mh = jnp.zeros((16,), jnp.float32) + jnp.max(
                jnp.where(iota16 == hg, mval, neg_inf))
            plsc.subcore_barrier()

            fill_fire(0, mh, hg, *bufs_a)

            def pair_body(k, c2, mh=mh, hg=hg):
                fill_fire(2 * k + 1, mh, hg, *bufs_b)
                drain_process(*bufs_a)
                fill_fire(2 * k + 2, mh, hg, *bufs_a)
                drain_process(*bufs_b)
                return c2

            lax.fori_loop(0, SB_PER_TILE // 2, pair_body, 0)
            pltpu.make_async_copy(whr.at[idx_a], rows_a, sem_a).wait()

            plsc.subcore_barrier()
            pltpu.sync_copy(acc.at[pl.ds(r0, ROWS_PER_TILE)],
                            out.at[2 * cid + hl, pl.ds(r0, ROWS_PER_TILE)])
            plsc.subcore_barrier()
            return carry

        lax.fori_loop(0, 2, head_body, 0)

    return edge_kernel


_EDGE_K0 = _make_edge_kernel(0)
_EDGE_K1 = _make_edge_kernel(1)


def _norm_body(acc_ref, out_ref):
    for h in range(H):
        a = acc_ref[h]
        out_ref[:, h * C:(h + 1) * C] = a[:, :C] / (a[:, C:C + 1] + 1e-16)


def _normalize(acc8):
    nblk = N_NODES // BN
    return pl.pallas_call(
        _norm_body,
        grid=(nblk,),
        in_specs=[pl.BlockSpec((H, BN, AW), lambda i: (0, i, 0))],
        out_specs=pl.BlockSpec((BN, H * C), lambda i: (i, 0)),
        out_shape=jax.ShapeDtypeStruct((N_NODES, H * C), jnp.float32),
    )(acc8)


def kernel(x, edge_index, W, a_left, a_right):
    W3 = W.reshape(D_IN, H, C)
    Wl = jnp.einsum("ihc,hc->ih", W3, a_left)
    Wr = jnp.einsum("ihc,hc->ih", W3, a_right)

    wh, el, er = _project(x, W, Wl, Wr)
    whr = wh.reshape(N_NODES * H, C)
    elt = el.T.reshape(H * N_NODES)
    ert = er.T.reshape(H * N_NODES)

    m = el.max(axis=0) + er.max(axis=0)
    m = jnp.where(m > 0, m, NEG * m)
    m16 = jnp.pad(m, (0, 16 - H)).astype(jnp.float32)

    loop = jnp.arange(N_NODES, dtype=jnp.int32)
    src = jnp.concatenate([edge_index[0].astype(jnp.int32), loop])
    dst = jnp.concatenate([edge_index[1].astype(jnp.int32), loop])
    srcp = jnp.pad(src, (0, E_PAD - E_TOT))
    dstp = jnp.pad(dst, (0, E_PAD - E_TOT))

    zrows = jnp.zeros((N_PAD, AW), jnp.float32)

    acc_lo = _EDGE_K0(whr, elt, ert, srcp, dstp, m16, zrows)
    acc_hi = _EDGE_K1(whr, elt, ert, srcp, dstp, m16, zrows)
    acc8 = jnp.concatenate([acc_lo, acc_hi], axis=0)[:, :N_NODES, :]

    return _normalize(acc8)
